# baseline (speedup 1.0000x reference)
import jax
import jax.numpy as jnp
from jax import lax
from jax.experimental import pallas as pl
from jax.experimental.pallas import tpu as pltpu
from jax.experimental.pallas import tpu_sc as plsc

N_NODES = 10000
N_PAD = 10240
E_PAD = 163840
CHUNK = 128
MBLK = 512
NROW = N_PAD // 16
NSUB = NROW // CHUNK


def _tc1_body(x_ref, w_ref, yself_ref, qab_ref):
    y = jnp.dot(x_ref[...], w_ref[...], preferred_element_type=jnp.float32)
    yself_ref[...] = y[:, :256]
    qab_ref[0] = y[:, 256:384]
    qab_ref[1] = y[:, 384:512]


def _tc1(xp, w1h):
    nb = N_PAD // MBLK
    return pl.pallas_call(
        _tc1_body,
        grid=(nb,),
        in_specs=[
            pl.BlockSpec((MBLK, 256), lambda m: (m, 0)),
            pl.BlockSpec((256, 512), lambda m: (0, 0)),
        ],
        out_specs=[
            pl.BlockSpec((MBLK, 256), lambda m: (m, 0)),
            pl.BlockSpec((2, MBLK, 128), lambda m: (0, m, 0)),
        ],
        out_shape=[
            jax.ShapeDtypeStruct((N_PAD, 256), jnp.float32),
            jax.ShapeDtypeStruct((2, N_PAD, 128), jnp.float32),
        ],
    )(xp, w1h)


def _tc2_body(yself_ref, s1_ref, d0_ref, d1_ref, b1_ref, w2h_ref,
              zself_ref, q2_ref):
    m = pl.program_id(0)
    deg = jnp.maximum(d0_ref[...] + d1_ref[...], 1.0)
    inv = 1.0 / deg
    h_n = jnp.concatenate([s1_ref[0], s1_ref[1]], axis=1)
    pre = yself_ref[...] + b1_ref[...] + h_n * inv
    row = m * MBLK + lax.broadcasted_iota(jnp.int32, (MBLK, 1), 0)
    out1 = jnp.where(row < N_NODES, jnp.maximum(pre, 0.0), 0.0)
    z = jnp.dot(out1, w2h_ref[...], preferred_element_type=jnp.float32)
    zself_ref[...] = z[:, :128]
    q2_ref[...] = z[:, 128:]


def _tc2(yself, s1, deg8, b1, w2h):
    nb = N_PAD // MBLK
    return pl.pallas_call(
        _tc2_body,
        grid=(nb,),
        in_specs=[
            pl.BlockSpec((MBLK, 256), lambda m: (m, 0)),
            pl.BlockSpec((2, MBLK, 128), lambda m: (0, m, 0)),
            pl.BlockSpec((MBLK, 1), lambda m: (m, 0)),
            pl.BlockSpec((MBLK, 1), lambda m: (m + nb, 0)),
            pl.BlockSpec((1, 256), lambda m: (0, 0)),
            pl.BlockSpec((256, 256), lambda m: (0, 0)),
        ],
        out_specs=[
            pl.BlockSpec((MBLK, 128), lambda m: (m, 0)),
            pl.BlockSpec((MBLK, 128), lambda m: (m, 0)),
        ],
        out_shape=[
            jax.ShapeDtypeStruct((N_PAD, 128), jnp.float32),
            jax.ShapeDtypeStruct((N_PAD, 128), jnp.float32),
        ],
    )(yself, s1, deg8, deg8, b1, w2h)


def _tc3_body(zself_ref, s2a_ref, s2b_ref, d0_ref, d1_ref, b2_ref, out_ref):
    deg = jnp.maximum(d0_ref[...] + d1_ref[...], 1.0)
    inv = 1.0 / deg
    s2 = s2a_ref[...] + s2b_ref[...]
    out_ref[...] = zself_ref[...] + b2_ref[...] + s2 * inv


def _tc3(zself, s2p, deg8, b2):
    nb = N_PAD // MBLK
    return pl.pallas_call(
        _tc3_body,
        grid=(nb,),
        in_specs=[
            pl.BlockSpec((MBLK, 128), lambda m: (m, 0)),
            pl.BlockSpec((MBLK, 128), lambda m: (m, 0)),
            pl.BlockSpec((MBLK, 128), lambda m: (m + nb, 0)),
            pl.BlockSpec((MBLK, 1), lambda m: (m, 0)),
            pl.BlockSpec((MBLK, 1), lambda m: (m + nb, 0)),
            pl.BlockSpec((1, 128), lambda m: (0, 0)),
        ],
        out_specs=pl.BlockSpec((MBLK, 128), lambda m: (m, 0)),
        out_shape=jax.ShapeDtypeStruct((N_PAD, 128), jnp.float32),
    )(zself, s2p, s2p, deg8, deg8, b2)


_MESH = dict(core_axis_name="c", subcore_axis_name="s",
             num_cores=2, num_subcores=16)

def _zero_vmem(ref):
    nrow, ncol = ref.shape

    def step(i, carry):
        r = i // (ncol // 16)
        c = lax.rem(i, ncol // 16) * 16
        ref[r, pl.ds(c, 16)] = jnp.zeros((16,), jnp.float32)
        return carry

    lax.fori_loop(0, nrow * (ncol // 16), step, 0, unroll=False)


P1 = 4
SLAB1 = 20
NDC1 = 10
SLAB2 = 40


def _agg_pass(qab, acc, sidx2, didx2, rowsA, rowsB, semA, semB, nbody):
    pltpu.async_copy(qab.at[sidx2.at[0]], rowsA, semA)

    def step(j, carry):
        c0 = 2 * j
        pltpu.async_copy(qab.at[sidx2.at[c0 + 1]], rowsB, semB)
        pltpu.make_async_copy(qab.at[pl.ds(0, CHUNK)], rowsA, semA).wait()
        pltpu.sync_copy(rowsA, acc.at[didx2.at[c0]], add=True)
        pltpu.async_copy(qab.at[sidx2.at[c0 + 2]], rowsA, semA)
        pltpu.make_async_copy(qab.at[pl.ds(0, CHUNK)], rowsB, semB).wait()
        pltpu.sync_copy(rowsB, acc.at[didx2.at[c0 + 1]], add=True)
        return carry

    lax.fori_loop(0, nbody, step, 0, unroll=False)
    pltpu.make_async_copy(qab.at[pl.ds(0, CHUNK)], rowsA, semA).wait()


def _sc1_body(qab, src5, dst5, dd5, me5,
              s1, degp1,
              sidx2, didx2, didx3, mval2, rowsA, rowsB,
              acc, dacc1, semA, semB, semD):
    cid = lax.axis_index("c")
    sid = lax.axis_index("s")
    wid = cid * 16 + sid
    _zero_vmem(rowsA)
    for j in range(NSUB):
        zs = pl.ds(sid * NROW + j * CHUNK, CHUNK)
        pltpu.sync_copy(rowsA, acc.at[zs])
        pltpu.sync_copy(rowsA.at[0], dacc1.at[zs])
    plsc.subcore_barrier()

    for p in range(P1):
        pltpu.sync_copy(src5.at[wid * P1 + p], sidx2)
        pltpu.sync_copy(dst5.at[sid * P1 + p], didx2)
        pltpu.sync_copy(dd5.at[wid * P1 + p], didx3)
        pltpu.sync_copy(me5.at[wid * P1 + p], mval2)
        for j in range(NDC1):
            pltpu.async_copy(mval2.at[j], dacc1.at[didx3.at[j]], semD,
                             add=True)
        _agg_pass(qab, acc, sidx2, didx2, rowsA, rowsB, semA, semB,
                  SLAB1 // 2)
        pltpu.make_async_copy(me5.at[wid * P1 + p], mval2, semD).wait()
    plsc.subcore_barrier()

    for j in range(NSUB):
        zs = pl.ds(sid * NROW + j * CHUNK, CHUNK)
        hs = pl.ds(cid * N_PAD + sid * NROW + j * CHUNK, CHUNK)
        pltpu.sync_copy(acc.at[zs], rowsA)
        pltpu.sync_copy(rowsA, s1.at[hs])
        pltpu.sync_copy(dacc1.at[zs], rowsB.at[0])
        pltpu.sync_copy(rowsB.at[0], degp1.at[hs])


def _sc1(qab, src5, dst5, dd5, me5):
    mesh = plsc.VectorSubcoreMesh(**_MESH)
    f = pl.kernel(
        _sc1_body,
        out_type=[
            jax.ShapeDtypeStruct((2 * N_PAD, 128), jnp.float32),
            jax.ShapeDtypeStruct((2 * N_PAD,), jnp.float32),
        ],
        mesh=mesh,
        scratch_types=[
            pltpu.VMEM((SLAB1 + 1, CHUNK), jnp.int32),
            pltpu.VMEM((SLAB1, CHUNK), jnp.int32),
            pltpu.VMEM((NDC1, CHUNK), jnp.int32),
            pltpu.VMEM((NDC1, CHUNK), jnp.float32),
            pltpu.VMEM((CHUNK, 128), jnp.float32),
            pltpu.VMEM((CHUNK, 128), jnp.float32),
            pltpu.VMEM_SHARED((N_PAD, 128), jnp.float32),
            pltpu.VMEM_SHARED((N_PAD,), jnp.float32),
            pltpu.SemaphoreType.DMA,
            pltpu.SemaphoreType.DMA,
            pltpu.SemaphoreType.DMA,
        ],
    )
    return f(qab, src5, dst5, dd5, me5)


def _sc2_body(q2, srcp5, dstp5,
              s2p,
              sidx2, didx2, rowsA, rowsB, acc, semA, semB):
    cid = lax.axis_index("c")
    sid = lax.axis_index("s")
    wid = cid * 16 + sid
    _zero_vmem(rowsA)
    for j in range(NSUB):
        zs = pl.ds(sid * NROW + j * CHUNK, CHUNK)
        pltpu.sync_copy(rowsA, acc.at[zs])
    plsc.subcore_barrier()

    pltpu.sync_copy(srcp5.at[wid], sidx2)
    pltpu.sync_copy(dstp5.at[wid], didx2)
    _agg_pass(q2, acc, sidx2, didx2, rowsA, rowsB, semA, semB, SLAB2 // 2)
    plsc.subcore_barrier()
    for j in range(NSUB):
        zs = pl.ds(sid * NROW + j * CHUNK, CHUNK)
        hs = pl.ds(cid * N_PAD + sid * NROW + j * CHUNK, CHUNK)
        pltpu.sync_copy(acc.at[zs], rowsA)
        pltpu.sync_copy(rowsA, s2p.at[hs])


def _sc2(q2, srcp5, dstp5):
    mesh = plsc.VectorSubcoreMesh(**_MESH)
    f = pl.kernel(
        _sc2_body,
        out_type=jax.ShapeDtypeStruct((2 * N_PAD, 128), jnp.float32),
        mesh=mesh,
        scratch_types=[
            pltpu.VMEM((SLAB2 + 1, CHUNK), jnp.int32),
            pltpu.VMEM((SLAB2, CHUNK), jnp.int32),
            pltpu.VMEM((CHUNK, 128), jnp.float32),
            pltpu.VMEM((CHUNK, 128), jnp.float32),
            pltpu.VMEM_SHARED((N_PAD, 128), jnp.float32),
            pltpu.SemaphoreType.DMA,
            pltpu.SemaphoreType.DMA,
        ],
    )
    return f(q2, srcp5, dstp5)


def kernel(x, edge_index, W1, b1, W2, b2):
    src = edge_index[0].astype(jnp.int32)
    dst = edge_index[1].astype(jnp.int32)
    npad = E_PAD - src.shape[0]
    srcp = jnp.pad(src, (0, npad), constant_values=N_NODES)
    dstp = jnp.pad(dst, (0, npad), constant_values=0)
    src2 = jnp.concatenate([srcp, srcp + N_PAD])
    emask1 = jnp.pad(jnp.ones((src.shape[0],), jnp.float32), (0, npad))
    pad128 = jnp.full((128, 1, CHUNK), N_NODES, jnp.int32)
    src5 = jnp.concatenate([src2.reshape(128, SLAB1, CHUNK), pad128], axis=1)
    dst5 = dstp.reshape(64, SLAB1, CHUNK)
    dd5 = dstp.reshape(128, NDC1, CHUNK)
    me5 = emask1.reshape(128, NDC1, CHUNK)
    srcp5 = jnp.concatenate(
        [srcp.reshape(32, SLAB2, CHUNK), pad128[:32]], axis=1)
    dst5b = dstp.reshape(32, SLAB2, CHUNK)

    xp = jnp.pad(x, ((0, N_PAD - N_NODES), (0, 0)))
    w1h = jnp.concatenate([W1[:256], W1[256:]], axis=1)
    w2h = jnp.concatenate([W2[:256], W2[256:]], axis=1)

    yself, qab = _tc1(xp, w1h)
    qab2 = qab.reshape(2 * N_PAD, 128)
    s1, deg1 = _sc1(qab2, src5, dst5, dd5, me5)
    s1 = s1.reshape(2, N_PAD, 128)
    degc = deg1.reshape(2 * N_PAD, 1)
    zself, q2 = _tc2(yself, s1, degc, b1.reshape(1, 256), w2h)
    s2p = _sc2(q2, srcp5, dst5b)
    out = _tc3(zself, s2p, degc, b2.reshape(1, 128))
    return out[:N_NODES]


# speedup vs baseline: 1.8220x; 1.8220x over previous
import jax
import jax.numpy as jnp
from jax import lax
from jax.experimental import pallas as pl
from jax.experimental.pallas import tpu as pltpu
from jax.experimental.pallas import tpu_sc as plsc

N_NODES = 10000
N_PAD = 10240
E_PAD = 163840
CHUNK = 128
MBLK = 512
NROW = N_PAD // 16
NSUB = NROW // CHUNK


def _tc1_body(x_ref, w_ref, yself_ref, qab_ref):
    y = jnp.dot(x_ref[...], w_ref[...], preferred_element_type=jnp.float32)
    yself_ref[...] = y[:, :256]
    qab_ref[0] = y[:, 256:384]
    qab_ref[1] = y[:, 384:512]


def _tc1(xp, w1h):
    nb = N_PAD // MBLK
    return pl.pallas_call(
        _tc1_body,
        grid=(nb,),
        in_specs=[
            pl.BlockSpec((MBLK, 256), lambda m: (m, 0)),
            pl.BlockSpec((256, 512), lambda m: (0, 0)),
        ],
        out_specs=[
            pl.BlockSpec((MBLK, 256), lambda m: (m, 0)),
            pl.BlockSpec((2, MBLK, 128), lambda m: (0, m, 0)),
        ],
        out_shape=[
            jax.ShapeDtypeStruct((N_PAD, 256), jnp.float32),
            jax.ShapeDtypeStruct((2, N_PAD, 128), jnp.float32),
        ],
    )(xp, w1h)


def _tc2_body(yself_ref, s1_ref, d0_ref, d1_ref, b1_ref, w2h_ref,
              zself_ref, q2_ref):
    m = pl.program_id(0)
    deg = jnp.maximum(d0_ref[...] + d1_ref[...], 1.0)
    inv = 1.0 / deg
    h_n = jnp.concatenate([s1_ref[0], s1_ref[1]], axis=1)
    pre = yself_ref[...] + b1_ref[...] + h_n * inv
    row = m * MBLK + lax.broadcasted_iota(jnp.int32, (MBLK, 1), 0)
    out1 = jnp.where(row < N_NODES, jnp.maximum(pre, 0.0), 0.0)
    z = jnp.dot(out1, w2h_ref[...], preferred_element_type=jnp.float32)
    zself_ref[...] = z[:, :128]
    q2_ref[...] = z[:, 128:]


def _tc2(yself, s1, deg8, b1, w2h):
    nb = N_PAD // MBLK
    return pl.pallas_call(
        _tc2_body,
        grid=(nb,),
        in_specs=[
            pl.BlockSpec((MBLK, 256), lambda m: (m, 0)),
            pl.BlockSpec((2, MBLK, 128), lambda m: (0, m, 0)),
            pl.BlockSpec((MBLK, 1), lambda m: (m, 0)),
            pl.BlockSpec((MBLK, 1), lambda m: (m + nb, 0)),
            pl.BlockSpec((1, 256), lambda m: (0, 0)),
            pl.BlockSpec((256, 256), lambda m: (0, 0)),
        ],
        out_specs=[
            pl.BlockSpec((MBLK, 128), lambda m: (m, 0)),
            pl.BlockSpec((MBLK, 128), lambda m: (m, 0)),
        ],
        out_shape=[
            jax.ShapeDtypeStruct((N_PAD, 128), jnp.float32),
            jax.ShapeDtypeStruct((N_PAD, 128), jnp.float32),
        ],
    )(yself, s1, deg8, deg8, b1, w2h)


def _tc3_body(zself_ref, s2a_ref, s2b_ref, d0_ref, d1_ref, b2_ref, out_ref):
    deg = jnp.maximum(d0_ref[...] + d1_ref[...], 1.0)
    inv = 1.0 / deg
    s2 = s2a_ref[...] + s2b_ref[...]
    out_ref[...] = zself_ref[...] + b2_ref[...] + s2 * inv


def _tc3(zself, s2p, deg8, b2):
    nb = N_PAD // MBLK
    return pl.pallas_call(
        _tc3_body,
        grid=(nb,),
        in_specs=[
            pl.BlockSpec((MBLK, 128), lambda m: (m, 0)),
            pl.BlockSpec((MBLK, 128), lambda m: (m, 0)),
            pl.BlockSpec((MBLK, 128), lambda m: (m + nb, 0)),
            pl.BlockSpec((MBLK, 1), lambda m: (m, 0)),
            pl.BlockSpec((MBLK, 1), lambda m: (m + nb, 0)),
            pl.BlockSpec((1, 128), lambda m: (0, 0)),
        ],
        out_specs=pl.BlockSpec((MBLK, 128), lambda m: (m, 0)),
        out_shape=jax.ShapeDtypeStruct((N_PAD, 128), jnp.float32),
    )(zself, s2p, s2p, deg8, deg8, b2)


_MESH = dict(core_axis_name="c", subcore_axis_name="s",
             num_cores=2, num_subcores=16)

def _zero_vmem(ref):
    nrow, ncol = ref.shape

    def step(i, carry):
        r = i // (ncol // 16)
        c = lax.rem(i, ncol // 16) * 16
        ref[r, pl.ds(c, 16)] = jnp.zeros((16,), jnp.float32)
        return carry

    lax.fori_loop(0, nrow * (ncol // 16), step, 0, unroll=False)


P1 = 2
SLAB1 = 40
NDC1 = 20
SLAB2 = 40


def _agg_pass(qab, acc, sidx2, didx2, rowsA, semA, nbody):
    def step(j, carry):
        pltpu.async_copy(qab.at[sidx2.at[j]], rowsA, semA).wait()
        pltpu.sync_copy(rowsA, acc.at[didx2.at[j]], add=True)
        return carry

    lax.fori_loop(0, nbody, step, 0, unroll=False)


def _sc1_body(qab, src5, dst5, dd5, me5,
              s1, degp1,
              sidx2, didx2, didx3, mval2, rowsA,
              acc, dacc1, semA):
    cid = lax.axis_index("c")
    sid = lax.axis_index("s")
    wid = cid * 16 + sid
    _zero_vmem(rowsA)
    for j in range(NSUB):
        zs = pl.ds(sid * NROW + j * CHUNK, CHUNK)
        pltpu.sync_copy(rowsA, acc.at[zs])
        pltpu.sync_copy(rowsA.at[0], dacc1.at[zs])
    plsc.subcore_barrier()

    for p in range(P1):
        pltpu.sync_copy(src5.at[wid * P1 + p], sidx2)
        pltpu.sync_copy(dst5.at[sid * P1 + p], didx2)
        pltpu.sync_copy(dd5.at[wid * P1 + p], didx3)
        pltpu.sync_copy(me5.at[wid * P1 + p], mval2)
        _agg_pass(qab, acc, sidx2, didx2, rowsA, semA, SLAB1)

        def dstep(j, carry):
            pltpu.sync_copy(mval2.at[j], dacc1.at[didx3.at[j]], add=True)
            return carry

        lax.fori_loop(0, NDC1, dstep, 0, unroll=False)
    plsc.subcore_barrier()

    for j in range(NSUB):
        zs = pl.ds(sid * NROW + j * CHUNK, CHUNK)
        hs = pl.ds(cid * N_PAD + sid * NROW + j * CHUNK, CHUNK)
        pltpu.sync_copy(acc.at[zs], rowsA)
        pltpu.sync_copy(rowsA, s1.at[hs])
        pltpu.sync_copy(dacc1.at[zs], rowsA.at[0])
        pltpu.sync_copy(rowsA.at[0], degp1.at[hs])


def _sc1(qab, src5, dst5, dd5, me5):
    mesh = plsc.VectorSubcoreMesh(**_MESH)
    f = pl.kernel(
        _sc1_body,
        out_type=[
            jax.ShapeDtypeStruct((2 * N_PAD, 128), jnp.float32),
            jax.ShapeDtypeStruct((2 * N_PAD,), jnp.float32),
        ],
        mesh=mesh,
        scratch_types=[
            pltpu.VMEM((SLAB1 + 1, CHUNK), jnp.int32),
            pltpu.VMEM((SLAB1, CHUNK), jnp.int32),
            pltpu.VMEM((NDC1, CHUNK), jnp.int32),
            pltpu.VMEM((NDC1, CHUNK), jnp.float32),
            pltpu.VMEM((CHUNK, 128), jnp.float32),
            pltpu.VMEM_SHARED((N_PAD, 128), jnp.float32),
            pltpu.VMEM_SHARED((N_PAD,), jnp.float32),
            pltpu.SemaphoreType.DMA,
        ],
    )
    return f(qab, src5, dst5, dd5, me5)


def _sc2_body(q2, srcp5, dstp5,
              s2p,
              sidx2, didx2, rowsA, acc, semA):
    cid = lax.axis_index("c")
    sid = lax.axis_index("s")
    wid = cid * 16 + sid
    _zero_vmem(rowsA)
    for j in range(NSUB):
        zs = pl.ds(sid * NROW + j * CHUNK, CHUNK)
        pltpu.sync_copy(rowsA, acc.at[zs])
    plsc.subcore_barrier()

    pltpu.sync_copy(srcp5.at[wid], sidx2)
    pltpu.sync_copy(dstp5.at[wid], didx2)
    _agg_pass(q2, acc, sidx2, didx2, rowsA, semA, SLAB2)
    plsc.subcore_barrier()
    for j in range(NSUB):
        zs = pl.ds(sid * NROW + j * CHUNK, CHUNK)
        hs = pl.ds(cid * N_PAD + sid * NROW + j * CHUNK, CHUNK)
        pltpu.sync_copy(acc.at[zs], rowsA)
        pltpu.sync_copy(rowsA, s2p.at[hs])


def _sc2(q2, srcp5, dstp5):
    mesh = plsc.VectorSubcoreMesh(**_MESH)
    f = pl.kernel(
        _sc2_body,
        out_type=jax.ShapeDtypeStruct((2 * N_PAD, 128), jnp.float32),
        mesh=mesh,
        scratch_types=[
            pltpu.VMEM((SLAB2 + 1, CHUNK), jnp.int32),
            pltpu.VMEM((SLAB2, CHUNK), jnp.int32),
            pltpu.VMEM((CHUNK, 128), jnp.float32),
            pltpu.VMEM_SHARED((N_PAD, 128), jnp.float32),
            pltpu.SemaphoreType.DMA,
        ],
    )
    return f(q2, srcp5, dstp5)


def kernel(x, edge_index, W1, b1, W2, b2):
    src = edge_index[0].astype(jnp.int32)
    dst = edge_index[1].astype(jnp.int32)
    npad = E_PAD - src.shape[0]
    srcp = jnp.pad(src, (0, npad), constant_values=N_NODES)
    dstp = jnp.pad(dst, (0, npad), constant_values=0)
    src2 = jnp.concatenate([srcp, srcp + N_PAD])
    emask1 = jnp.pad(jnp.ones((src.shape[0],), jnp.float32), (0, npad))
    pad1 = jnp.full((32 * P1, 1, CHUNK), N_NODES, jnp.int32)
    src5 = jnp.concatenate(
        [src2.reshape(32 * P1, SLAB1, CHUNK), pad1], axis=1)
    dst5 = dstp.reshape(16 * P1, SLAB1, CHUNK)
    dd5 = dstp.reshape(32 * P1, NDC1, CHUNK)
    me5 = emask1.reshape(32 * P1, NDC1, CHUNK)
    srcp5 = jnp.concatenate(
        [srcp.reshape(32, SLAB2, CHUNK), pad1[:32]], axis=1)
    dst5b = dstp.reshape(32, SLAB2, CHUNK)

    xp = jnp.pad(x, ((0, N_PAD - N_NODES), (0, 0)))
    w1h = jnp.concatenate([W1[:256], W1[256:]], axis=1)
    w2h = jnp.concatenate([W2[:256], W2[256:]], axis=1)

    yself, qab = _tc1(xp, w1h)
    qab2 = qab.reshape(2 * N_PAD, 128)
    s1, deg1 = _sc1(qab2, src5, dst5, dd5, me5)
    s1 = s1.reshape(2, N_PAD, 128)
    degc = deg1.reshape(2 * N_PAD, 1)
    zself, q2 = _tc2(yself, s1, degc, b1.reshape(1, 256), w2h)
    s2p = _sc2(q2, srcp5, dst5b)
    out = _tc3(zself, s2p, degc, b2.reshape(1, 128))
    return out[:N_NODES]


# speedup vs baseline: 1.9640x; 1.0779x over previous
import jax
import jax.numpy as jnp
from jax import lax
from jax.experimental import pallas as pl
from jax.experimental.pallas import tpu as pltpu
from jax.experimental.pallas import tpu_sc as plsc

N_NODES = 10000
N_PAD = 10240
E_PAD = 163840
CHUNK = 128
MBLK = 512
NROW = N_PAD // 16
NSUB = NROW // CHUNK


def _tc1_body(x_ref, w_ref, yself_ref, qab_ref):
    y = jnp.dot(x_ref[...], w_ref[...], preferred_element_type=jnp.float32)
    yself_ref[...] = y[:, :256]
    qab_ref[0] = y[:, 256:384]
    qab_ref[1] = y[:, 384:512]


def _tc1(xp, w1h):
    nb = N_PAD // MBLK
    return pl.pallas_call(
        _tc1_body,
        grid=(nb,),
        in_specs=[
            pl.BlockSpec((MBLK, 256), lambda m: (m, 0)),
            pl.BlockSpec((256, 512), lambda m: (0, 0)),
        ],
        out_specs=[
            pl.BlockSpec((MBLK, 256), lambda m: (m, 0)),
            pl.BlockSpec((2, MBLK, 128), lambda m: (0, m, 0)),
        ],
        out_shape=[
            jax.ShapeDtypeStruct((N_PAD, 256), jnp.float32),
            jax.ShapeDtypeStruct((2, N_PAD, 128), jnp.float32),
        ],
    )(xp, w1h)


def _tc2_body(yself_ref, s1_ref, d0_ref, d1_ref, b1_ref, w2h_ref,
              zself_ref, q2_ref):
    m = pl.program_id(0)
    deg = jnp.maximum(d0_ref[...] + d1_ref[...], 1.0)
    inv = 1.0 / deg
    h_n = jnp.concatenate([s1_ref[0], s1_ref[1]], axis=1)
    pre = yself_ref[...] + b1_ref[...] + h_n * inv
    row = m * MBLK + lax.broadcasted_iota(jnp.int32, (MBLK, 1), 0)
    out1 = jnp.where(row < N_NODES, jnp.maximum(pre, 0.0), 0.0)
    z = jnp.dot(out1, w2h_ref[...], preferred_element_type=jnp.float32)
    zself_ref[...] = z[:, :128]
    q2_ref[...] = z[:, 128:]


def _tc2(yself, s1, deg8, b1, w2h):
    nb = N_PAD // MBLK
    return pl.pallas_call(
        _tc2_body,
        grid=(nb,),
        in_specs=[
            pl.BlockSpec((MBLK, 256), lambda m: (m, 0)),
            pl.BlockSpec((2, MBLK, 128), lambda m: (0, m, 0)),
            pl.BlockSpec((MBLK, 1), lambda m: (m, 0)),
            pl.BlockSpec((MBLK, 1), lambda m: (m + nb, 0)),
            pl.BlockSpec((1, 256), lambda m: (0, 0)),
            pl.BlockSpec((256, 256), lambda m: (0, 0)),
        ],
        out_specs=[
            pl.BlockSpec((MBLK, 128), lambda m: (m, 0)),
            pl.BlockSpec((MBLK, 128), lambda m: (m, 0)),
        ],
        out_shape=[
            jax.ShapeDtypeStruct((N_PAD, 128), jnp.float32),
            jax.ShapeDtypeStruct((N_PAD, 128), jnp.float32),
        ],
    )(yself, s1, deg8, deg8, b1, w2h)


def _tc3_body(zself_ref, s2a_ref, s2b_ref, d0_ref, d1_ref, b2_ref, out_ref):
    deg = jnp.maximum(d0_ref[...] + d1_ref[...], 1.0)
    inv = 1.0 / deg
    s2 = s2a_ref[...] + s2b_ref[...]
    out_ref[...] = zself_ref[...] + b2_ref[...] + s2 * inv


def _tc3(zself, s2p, deg8, b2):
    nb = N_PAD // MBLK
    return pl.pallas_call(
        _tc3_body,
        grid=(nb,),
        in_specs=[
            pl.BlockSpec((MBLK, 128), lambda m: (m, 0)),
            pl.BlockSpec((MBLK, 128), lambda m: (m, 0)),
            pl.BlockSpec((MBLK, 128), lambda m: (m + nb, 0)),
            pl.BlockSpec((MBLK, 1), lambda m: (m, 0)),
            pl.BlockSpec((MBLK, 1), lambda m: (m + nb, 0)),
            pl.BlockSpec((1, 128), lambda m: (0, 0)),
        ],
        out_specs=pl.BlockSpec((MBLK, 128), lambda m: (m, 0)),
        out_shape=jax.ShapeDtypeStruct((N_PAD, 128), jnp.float32),
    )(zself, s2p, s2p, deg8, deg8, b2)


_MESH = dict(core_axis_name="c", subcore_axis_name="s",
             num_cores=2, num_subcores=16)

def _zero_vmem(ref):
    nrow, ncol = ref.shape

    def step(i, carry):
        r = i // (ncol // 16)
        c = lax.rem(i, ncol // 16) * 16
        ref[r, pl.ds(c, 16)] = jnp.zeros((16,), jnp.float32)
        return carry

    lax.fori_loop(0, nrow * (ncol // 16), step, 0, unroll=False)


P1 = 2
SLAB1 = 40
NDC1 = 20
SLAB2 = 40


def _agg_pass(qab, acc, sidx2, didx2, rowsA, semA, nbody):
    def step(j, carry):
        pltpu.async_copy(qab.at[sidx2.at[j]], rowsA, semA).wait()
        pltpu.sync_copy(rowsA, acc.at[didx2.at[j]], add=True)
        return carry

    lax.fori_loop(0, nbody, step, 0, unroll=False)


def _sc1_body(qab, src5, dst5, dd5, me5,
              s1, degp1,
              sidx2, didx2, didx3, mval2, rowsA,
              acc, dacc1, semA):
    cid = lax.axis_index("c")
    sid = lax.axis_index("s")
    wid = cid * 16 + sid
    _zero_vmem(rowsA)
    for j in range(NSUB):
        zs = pl.ds(sid * NROW + j * CHUNK, CHUNK)
        pltpu.sync_copy(rowsA, acc.at[zs])
        pltpu.sync_copy(rowsA.at[0], dacc1.at[zs])
    plsc.subcore_barrier()

    for p in range(P1):
        pltpu.sync_copy(src5.at[wid * P1 + p], sidx2)
        pltpu.sync_copy(dst5.at[sid * P1 + p], didx2)
        pltpu.sync_copy(dd5.at[wid * P1 + p], didx3)
        pltpu.sync_copy(me5.at[wid * P1 + p], mval2)
        _agg_pass(qab, acc, sidx2, didx2, rowsA, semA, SLAB1)

        def dstep(j, carry):
            pltpu.sync_copy(mval2.at[j], dacc1.at[didx3.at[j]], add=True)
            return carry

        lax.fori_loop(0, NDC1, dstep, 0, unroll=False)
    plsc.subcore_barrier()

    for j in range(NSUB):
        zs = pl.ds(sid * NROW + j * CHUNK, CHUNK)
        hs = pl.ds(cid * N_PAD + sid * NROW + j * CHUNK, CHUNK)
        pltpu.sync_copy(acc.at[zs], rowsA)
        pltpu.sync_copy(rowsA, s1.at[hs])
        pltpu.sync_copy(dacc1.at[zs], rowsA.at[0])
        pltpu.sync_copy(rowsA.at[0], degp1.at[hs])


def _sc1(qab, src5, dst5, dd5, me5):
    mesh = plsc.VectorSubcoreMesh(**_MESH)
    f = pl.kernel(
        _sc1_body,
        out_type=[
            jax.ShapeDtypeStruct((2 * N_PAD, 128), jnp.float32),
            jax.ShapeDtypeStruct((2 * N_PAD,), jnp.float32),
        ],
        mesh=mesh,
        scratch_types=[
            pltpu.VMEM((SLAB1 + 1, CHUNK), jnp.int32),
            pltpu.VMEM((SLAB1, CHUNK), jnp.int32),
            pltpu.VMEM((NDC1, CHUNK), jnp.int32),
            pltpu.VMEM((NDC1, CHUNK), jnp.float32),
            pltpu.VMEM((CHUNK, 128), jnp.float32),
            pltpu.VMEM_SHARED((N_PAD, 128), jnp.float32),
            pltpu.VMEM_SHARED((N_PAD,), jnp.float32),
            pltpu.SemaphoreType.DMA,
        ],
    )
    return f(qab, src5, dst5, dd5, me5)


def _sc2_body(q2, srcp5, dstp5,
              s2p,
              sidx2, didx2, rowsA, acc, semA):
    cid = lax.axis_index("c")
    sid = lax.axis_index("s")
    wid = cid * 16 + sid
    _zero_vmem(rowsA)
    for j in range(NSUB):
        zs = pl.ds(sid * NROW + j * CHUNK, CHUNK)
        pltpu.sync_copy(rowsA, acc.at[zs])
    plsc.subcore_barrier()

    pltpu.sync_copy(srcp5.at[wid], sidx2)
    pltpu.sync_copy(dstp5.at[wid], didx2)
    _agg_pass(q2, acc, sidx2, didx2, rowsA, semA, SLAB2)
    plsc.subcore_barrier()
    for j in range(NSUB):
        zs = pl.ds(sid * NROW + j * CHUNK, CHUNK)
        hs = pl.ds(cid * N_PAD + sid * NROW + j * CHUNK, CHUNK)
        pltpu.sync_copy(acc.at[zs], rowsA)
        pltpu.sync_copy(rowsA, s2p.at[hs])


def _sc2(q2, srcp5, dstp5):
    mesh = plsc.VectorSubcoreMesh(**_MESH)
    f = pl.kernel(
        _sc2_body,
        out_type=jax.ShapeDtypeStruct((2 * N_PAD, 128), jnp.float32),
        mesh=mesh,
        scratch_types=[
            pltpu.VMEM((SLAB2 + 1, CHUNK), jnp.int32),
            pltpu.VMEM((SLAB2, CHUNK), jnp.int32),
            pltpu.VMEM((CHUNK, 128), jnp.float32),
            pltpu.VMEM_SHARED((N_PAD, 128), jnp.float32),
            pltpu.SemaphoreType.DMA,
        ],
    )
    return f(q2, srcp5, dstp5)


def kernel(x, edge_index, W1, b1, W2, b2):
    src = edge_index[0].astype(jnp.int32)
    dst = edge_index[1].astype(jnp.int32)
    npad = E_PAD - src.shape[0]
    srcp = jnp.pad(src, (0, npad), constant_values=N_NODES)
    dstp = jnp.concatenate([dst, jnp.arange(npad, dtype=jnp.int32)])
    src2 = jnp.concatenate([srcp, srcp + N_PAD])
    emask1 = jnp.pad(jnp.ones((src.shape[0],), jnp.float32), (0, npad))
    pad1 = jnp.full((32 * P1, 1, CHUNK), N_NODES, jnp.int32)
    src5 = jnp.concatenate(
        [src2.reshape(32 * P1, SLAB1, CHUNK), pad1], axis=1)
    dst5 = dstp.reshape(16 * P1, SLAB1, CHUNK)
    dd5 = dstp.reshape(32 * P1, NDC1, CHUNK)
    me5 = emask1.reshape(32 * P1, NDC1, CHUNK)
    srcp5 = jnp.concatenate(
        [srcp.reshape(32, SLAB2, CHUNK), pad1[:32]], axis=1)
    dst5b = dstp.reshape(32, SLAB2, CHUNK)

    xp = jnp.pad(x, ((0, N_PAD - N_NODES), (0, 0)))
    w1h = jnp.concatenate([W1[:256], W1[256:]], axis=1)
    w2h = jnp.concatenate([W2[:256], W2[256:]], axis=1)

    yself, qab = _tc1(xp, w1h)
    qab2 = qab.reshape(2 * N_PAD, 128)
    s1, deg1 = _sc1(qab2, src5, dst5, dd5, me5)
    s1 = s1.reshape(2, N_PAD, 128)
    degc = deg1.reshape(2 * N_PAD, 1)
    zself, q2 = _tc2(yself, s1, degc, b1.reshape(1, 256), w2h)
    s2p = _sc2(q2, srcp5, dst5b)
    out = _tc3(zself, s2p, degc, b2.reshape(1, 128))
    return out[:N_NODES]
